# baseline (speedup 1.0000x reference)
import functools

import jax
import jax.numpy as jnp
from jax.experimental import pallas as pl
from jax.experimental.pallas import tpu as pltpu
from jax.experimental.pallas import tpu_sc as plsc

RATE = 0.7
ALPHA = 0.8
BETA = 1.2

SC_GROUP = 8
SC_NBUF = 16
SC_WORKERS_PER_ROW = 8


def _tc_sum_kernel(x_ref, o_ref, acc_ref, *, n_chunks):
    ci = pl.program_id(0)

    @pl.when(ci == 0)
    def _init():
        acc_ref[...] = jnp.zeros_like(acc_ref)

    acc_ref[...] += jnp.sum(x_ref[...], axis=1)

    @pl.when(ci == n_chunks - 1)
    def _finish():
        o_ref[...] = acc_ref[...]


def _sc_sum_body(x_hbm, out_hbm, buf_v, acc_v, sems, *, b, c0, c, chunk, nvec):
    cid = jax.lax.axis_index("c")
    sid = jax.lax.axis_index("s")
    wid = sid * 2 + cid
    bid = wid // SC_WORKERS_PER_ROW
    col0 = (wid % SC_WORKERS_PER_ROW) * chunk
    n_groups = (c - c0) // SC_GROUP

    @pl.when(bid < b)
    def _worker():
        _sc_sum_work(
            x_hbm, out_hbm, buf_v, acc_v, sems, bid, col0,
            c0=c0, c=c, chunk=chunk, nvec=nvec, n_groups=n_groups,
        )


def _sc_sum_work(
    x_hbm, out_hbm, buf_v, acc_v, sems, bid, col0,
    *, c0, c, chunk, nvec, n_groups
):

    def dma(ch, slot):
        return pltpu.make_async_copy(
            x_hbm.at[bid, ch, pl.ds(col0, chunk)],
            buf_v.at[slot],
            sems.at[slot],
        )

    def issue_group(g):
        base = jax.lax.rem(g, 2) * SC_GROUP
        for t in range(SC_GROUP):
            dma(c0 + g * SC_GROUP + t, base + t).start()

    issue_group(0)
    issue_group(1)

    def zbody(i, carry):
        acc_v[pl.ds(i * 16, 16)] = jnp.zeros((16,), jnp.float32)
        return carry

    jax.lax.fori_loop(0, nvec, zbody, 0)

    def gbody(g, carry):
        base = jax.lax.rem(g, 2) * SC_GROUP
        for t in range(SC_GROUP):
            dma(c0 + g * SC_GROUP + t, base + t).wait()

        def abody(i, carry):
            sl = pl.ds(i * 16, 16)
            s = acc_v[sl]
            for t in range(SC_GROUP):
                s = s + buf_v[base + t, sl]
            acc_v[sl] = s
            return carry

        jax.lax.fori_loop(0, nvec, abody, 0)

        @pl.when(g + 2 < n_groups)
        def _():
            issue_group(g + 2)

        return carry

    jax.lax.fori_loop(0, n_groups, gbody, 0)
    pltpu.sync_copy(acc_v, out_hbm.at[bid, pl.ds(col0, chunk)])


def _select_kernel(a_ref, b_ref, o_ref, *, k):
    h = a_ref[...] + b_ref[...]
    i32 = jax.lax.bitcast_convert_type(h, jnp.int32)
    key = jnp.where(i32 < 0, i32 ^ 0x7FFFFFFF, i32)
    ukey = jax.lax.bitcast_convert_type(key, jnp.uint32) ^ jnp.uint32(0x80000000)

    def body(t, T):
        bit = jnp.uint32(31 - t)
        cand = T | (jnp.uint32(1) << bit)
        cnt = jnp.sum((ukey >= cand).astype(jnp.int32), axis=1, keepdims=True)
        return jnp.where(cnt >= k, cand, T)

    T = jax.lax.fori_loop(0, 32, body, jnp.zeros((h.shape[0], 1), jnp.uint32))

    kk = jax.lax.bitcast_convert_type(T ^ jnp.uint32(0x80000000), jnp.int32)
    iv = jnp.where(kk < 0, kk ^ 0x7FFFFFFF, kk)
    v = jax.lax.bitcast_convert_type(iv, jnp.float32)
    o_ref[...] = jnp.where(h >= v, jnp.float32(ALPHA), jnp.float32(BETA))


def kernel(inputs):
    b, c, h, w = inputs.shape
    hw = h * w
    lanes = 128
    rows = hw // lanes
    k = int(RATE * hw)
    cc = 16
    csc = 2 * SC_GROUP if c >= 768 else max(2 * SC_GROUP, (c // 3) & ~(SC_GROUP - 1))
    c0 = (c - csc) // cc * cc
    n_chunks = c0 // cc
    chunk = hw // SC_WORKERS_PER_ROW
    nvec = chunk // 16

    x = inputs.reshape(b, c, rows, lanes)
    tc_sum = pl.pallas_call(
        functools.partial(_tc_sum_kernel, n_chunks=n_chunks),
        grid=(n_chunks,),
        in_specs=[pl.BlockSpec((b, cc, rows, lanes), lambda j: (0, j, 0, 0))],
        out_specs=pl.BlockSpec((b, rows, lanes), lambda j: (0, 0, 0)),
        out_shape=jax.ShapeDtypeStruct((b, rows, lanes), jnp.float32),
        scratch_shapes=[pltpu.VMEM((b, rows, lanes), jnp.float32)],
    )(x)

    sc_fn = pl.kernel(
        functools.partial(_sc_sum_body, b=b, c0=c0, c=c, chunk=chunk, nvec=nvec),
        out_type=jax.ShapeDtypeStruct((b, hw), jnp.float32),
        mesh=plsc.VectorSubcoreMesh(core_axis_name="c", subcore_axis_name="s"),
        scratch_types=[
            pltpu.VMEM((SC_NBUF, chunk), jnp.float32),
            pltpu.VMEM((chunk,), jnp.float32),
            pltpu.SemaphoreType.DMA((SC_NBUF,)),
        ],
    )
    sc_sum = sc_fn(inputs.reshape(b, c, hw))

    out = pl.pallas_call(
        functools.partial(_select_kernel, k=k),
        in_specs=[
            pl.BlockSpec((b, hw), lambda: (0, 0)),
            pl.BlockSpec((b, hw), lambda: (0, 0)),
        ],
        out_specs=pl.BlockSpec((b, hw), lambda: (0, 0)),
        out_shape=jax.ShapeDtypeStruct((b, hw), jnp.float32),
    )(tc_sum.reshape(b, hw), sc_sum)
    return out.reshape(b, 1, h, w)


# speedup vs baseline: 1.7306x; 1.7306x over previous
import functools

import jax
import jax.numpy as jnp
from jax.experimental import pallas as pl
from jax.experimental.pallas import tpu as pltpu

RATE = 0.7
ALPHA = 0.8
BETA = 1.2

NBUF = 4


def _mean_kernel(x_hbm, o_ref, acc_ref, *bufs_and_sems, n_chunks, cc):
    bufs = bufs_and_sems[:NBUF]
    sems = bufs_and_sems[NBUF:]

    def copy(i, slot):
        return pltpu.make_async_copy(
            x_hbm.at[:, pl.ds(i * cc, cc)], bufs[slot], sems[slot]
        )

    for s in range(min(NBUF, n_chunks)):
        copy(s, s).start()

    acc_ref[...] = jnp.zeros_like(acc_ref)

    for i in range(n_chunks):
        slot = i % NBUF
        copy(i, slot).wait()
        acc_ref[...] += jnp.sum(bufs[slot][...], axis=1)
        nxt = i + NBUF
        if nxt < n_chunks:
            copy(nxt, slot).start()

    o_ref[...] = acc_ref[...]


def _select_kernel(h_ref, o_ref, *, k):
    h = h_ref[...]
    i32 = jax.lax.bitcast_convert_type(h, jnp.int32)
    key = jnp.where(i32 < 0, i32 ^ 0x7FFFFFFF, i32)
    ukey = jax.lax.bitcast_convert_type(key, jnp.uint32) ^ jnp.uint32(0x80000000)

    def body(t, T):
        bit = jnp.uint32(31 - t)
        cand = T | (jnp.uint32(1) << bit)
        cnt = jnp.sum((ukey >= cand).astype(jnp.int32), axis=1, keepdims=True)
        return jnp.where(cnt >= k, cand, T)

    T = jax.lax.fori_loop(0, 32, body, jnp.zeros((h.shape[0], 1), jnp.uint32))

    kk = jax.lax.bitcast_convert_type(T ^ jnp.uint32(0x80000000), jnp.int32)
    iv = jnp.where(kk < 0, kk ^ 0x7FFFFFFF, kk)
    v = jax.lax.bitcast_convert_type(iv, jnp.float32)
    o_ref[...] = jnp.where(h >= v, jnp.float32(ALPHA), jnp.float32(BETA))


def kernel(inputs):
    b, c, h, w = inputs.shape
    hw = h * w
    lanes = 128
    rows = hw // lanes
    k = int(RATE * hw)
    cc = 8
    n_chunks = c // cc
    x = inputs.reshape(b, c, rows, lanes)
    heat = pl.pallas_call(
        functools.partial(_mean_kernel, n_chunks=n_chunks, cc=cc),
        in_specs=[pl.BlockSpec(memory_space=pltpu.HBM)],
        out_specs=pl.BlockSpec((b, rows, lanes), lambda: (0, 0, 0)),
        out_shape=jax.ShapeDtypeStruct((b, rows, lanes), jnp.float32),
        scratch_shapes=[pltpu.VMEM((b, rows, lanes), jnp.float32)]
        + [pltpu.VMEM((b, cc, rows, lanes), jnp.float32) for _ in range(NBUF)]
        + [pltpu.SemaphoreType.DMA for _ in range(NBUF)],
    )(x)
    out = pl.pallas_call(
        functools.partial(_select_kernel, k=k),
        in_specs=[pl.BlockSpec((b, hw), lambda: (0, 0))],
        out_specs=pl.BlockSpec((b, hw), lambda: (0, 0)),
        out_shape=jax.ShapeDtypeStruct((b, hw), jnp.float32),
    )(heat.reshape(b, hw))
    return out.reshape(b, 1, h, w)


# speedup vs baseline: 1.7340x; 1.0020x over previous
import functools

import jax
import jax.numpy as jnp
from jax.experimental import pallas as pl
from jax.experimental.pallas import tpu as pltpu

RATE = 0.7
ALPHA = 0.8
BETA = 1.2

NBUF = 4


def _mean_kernel(
    xb_ref, x_hbm, o_ref, acc_ref, buf_ref, sem_ref, *, n_steps, cc, c_half
):
    ci = pl.program_id(0)

    def copy(i, slot):
        return pltpu.make_async_copy(
            x_hbm.at[:, pl.ds(c_half + i * cc, cc)],
            buf_ref.at[slot],
            sem_ref.at[slot],
        )

    @pl.when(ci == 0)
    def _init():
        acc_ref[...] = jnp.zeros_like(acc_ref)
        for s in range(min(NBUF, n_steps)):
            copy(s, s).start()

    slot = jax.lax.rem(ci, NBUF)
    copy(ci, slot).wait()
    acc_ref[...] += jnp.sum(xb_ref[...], axis=1) + jnp.sum(
        buf_ref[slot], axis=1
    )
    nxt = ci + NBUF

    @pl.when(nxt < n_steps)
    def _refill():
        copy(nxt, slot).start()

    @pl.when(ci == n_steps - 1)
    def _finish():
        o_ref[...] = acc_ref[...]


def _select_kernel(h_ref, o_ref, *, k):
    h = h_ref[...]
    i32 = jax.lax.bitcast_convert_type(h, jnp.int32)
    key = jnp.where(i32 < 0, i32 ^ 0x7FFFFFFF, i32)
    ukey = jax.lax.bitcast_convert_type(key, jnp.uint32) ^ jnp.uint32(0x80000000)

    def body(t, T):
        bit = jnp.uint32(31 - t)
        cand = T | (jnp.uint32(1) << bit)
        cnt = jnp.sum((ukey >= cand).astype(jnp.int32), axis=1, keepdims=True)
        return jnp.where(cnt >= k, cand, T)

    T = jax.lax.fori_loop(0, 32, body, jnp.zeros((h.shape[0], 1), jnp.uint32))

    kk = jax.lax.bitcast_convert_type(T ^ jnp.uint32(0x80000000), jnp.int32)
    iv = jnp.where(kk < 0, kk ^ 0x7FFFFFFF, kk)
    v = jax.lax.bitcast_convert_type(iv, jnp.float32)
    o_ref[...] = jnp.where(h >= v, jnp.float32(ALPHA), jnp.float32(BETA))


def kernel(inputs):
    b, c, h, w = inputs.shape
    hw = h * w
    lanes = 128
    rows = hw // lanes
    k = int(RATE * hw)
    cc = 8
    c_half = c // 2
    n_steps = c_half // cc
    x = inputs.reshape(b, c, rows, lanes)
    heat = pl.pallas_call(
        functools.partial(
            _mean_kernel, n_steps=n_steps, cc=cc, c_half=c_half
        ),
        grid=(n_steps,),
        in_specs=[
            pl.BlockSpec((b, cc, rows, lanes), lambda j: (0, j, 0, 0)),
            pl.BlockSpec(memory_space=pltpu.HBM),
        ],
        out_specs=pl.BlockSpec((b, rows, lanes), lambda j: (0, 0, 0)),
        out_shape=jax.ShapeDtypeStruct((b, rows, lanes), jnp.float32),
        scratch_shapes=[
            pltpu.VMEM((b, rows, lanes), jnp.float32),
            pltpu.VMEM((NBUF, b, cc, rows, lanes), jnp.float32),
            pltpu.SemaphoreType.DMA((NBUF,)),
        ],
    )(x, x)
    out = pl.pallas_call(
        functools.partial(_select_kernel, k=k),
        in_specs=[pl.BlockSpec((b, hw), lambda: (0, 0))],
        out_specs=pl.BlockSpec((b, hw), lambda: (0, 0)),
        out_shape=jax.ShapeDtypeStruct((b, hw), jnp.float32),
    )(heat.reshape(b, hw))
    return out.reshape(b, 1, h, w)


# speedup vs baseline: 1.7650x; 1.0178x over previous
import functools

import jax
import jax.numpy as jnp
from jax.experimental import pallas as pl
from jax.experimental.pallas import tpu as pltpu

RATE = 0.7
ALPHA = 0.8
BETA = 1.2

NBUF = 16


def _fused_kernel(x_hbm, o_ref, acc_ref, buf_ref, sem_ref, *, n_chunks, cc, k):
    b = x_hbm.shape[0]
    bi = pl.program_id(0)
    total = b * n_chunks

    def copy(g, slot):
        return pltpu.make_async_copy(
            x_hbm.at[g // n_chunks, pl.ds(jax.lax.rem(g, n_chunks) * cc, cc)],
            buf_ref.at[slot],
            sem_ref.at[slot],
        )

    @pl.when(bi == 0)
    def _prime():
        for s in range(min(NBUF, total)):
            copy(s, s).start()

    acc_ref[...] = jnp.zeros_like(acc_ref)

    def body(j, carry):
        g = bi * n_chunks + j
        slot = jax.lax.rem(g, NBUF)
        copy(g, slot).wait()
        acc_ref[...] += jnp.sum(buf_ref[slot], axis=0)
        nxt = g + NBUF

        @pl.when(nxt < total)
        def _refill():
            copy(nxt, slot).start()

        return carry

    jax.lax.fori_loop(0, n_chunks, body, 0)

    h = acc_ref[...]
    i32 = jax.lax.bitcast_convert_type(h, jnp.int32)
    key = jnp.where(i32 < 0, i32 ^ 0x7FFFFFFF, i32)
    ukey = jax.lax.bitcast_convert_type(key, jnp.uint32) ^ jnp.uint32(0x80000000)

    def sbody(t, T):
        bit = jnp.uint32(31 - t)
        cand = T | (jnp.uint32(1) << bit)
        cnt = jnp.sum((ukey >= cand).astype(jnp.int32))
        return jnp.where(cnt >= k, cand, T)

    T = jax.lax.fori_loop(0, 32, sbody, jnp.uint32(0))

    kk = jax.lax.bitcast_convert_type(T ^ jnp.uint32(0x80000000), jnp.int32)
    iv = jnp.where(kk < 0, kk ^ 0x7FFFFFFF, kk)
    v = jax.lax.bitcast_convert_type(iv, jnp.float32)
    o_ref[0] = jnp.where(h >= v, jnp.float32(ALPHA), jnp.float32(BETA))


def kernel(inputs):
    b, c, h, w = inputs.shape
    hw = h * w
    lanes = 128
    rows = hw // lanes
    k = int(RATE * hw)
    cc = 8
    n_chunks = c // cc
    x = inputs.reshape(b, c, rows, lanes)
    out = pl.pallas_call(
        functools.partial(_fused_kernel, n_chunks=n_chunks, cc=cc, k=k),
        grid=(b,),
        in_specs=[pl.BlockSpec(memory_space=pltpu.HBM)],
        out_specs=pl.BlockSpec((1, rows, lanes), lambda i: (i, 0, 0)),
        out_shape=jax.ShapeDtypeStruct((b, rows, lanes), jnp.float32),
        scratch_shapes=[
            pltpu.VMEM((rows, lanes), jnp.float32),
            pltpu.VMEM((NBUF, cc, rows, lanes), jnp.float32),
            pltpu.SemaphoreType.DMA((NBUF,)),
        ],
    )(x)
    return out.reshape(b, 1, h, w)


# speedup vs baseline: 1.7719x; 1.0039x over previous
import functools

import jax
import jax.numpy as jnp
from jax.experimental import pallas as pl
from jax.experimental.pallas import tpu as pltpu

RATE = 0.7
ALPHA = 0.8
BETA = 1.2

NBUF = 8


def _fused_kernel(x_hbm, o_ref, acc_ref, buf_ref, sem_ref, *, n_chunks, cc, k):
    b = x_hbm.shape[0]
    bi = pl.program_id(0)
    total = b * n_chunks

    def copy(g, slot):
        return pltpu.make_async_copy(
            x_hbm.at[g // n_chunks, pl.ds(jax.lax.rem(g, n_chunks) * cc, cc)],
            buf_ref.at[slot],
            sem_ref.at[slot],
        )

    @pl.when(bi == 0)
    def _prime():
        for s in range(min(NBUF, total)):
            copy(s, s).start()

    acc_ref[...] = jnp.zeros_like(acc_ref)

    def body(j, carry):
        g = bi * n_chunks + j
        slot = jax.lax.rem(g, NBUF)
        copy(g, slot).wait()
        acc_ref[...] += jnp.sum(buf_ref[slot], axis=0)
        nxt = g + NBUF

        @pl.when(nxt < total)
        def _refill():
            copy(nxt, slot).start()

        return carry

    jax.lax.fori_loop(0, n_chunks, body, 0)

    h = acc_ref[...]
    i32 = jax.lax.bitcast_convert_type(h, jnp.int32)
    key = jnp.where(i32 < 0, i32 ^ 0x7FFFFFFF, i32)
    ukey = jax.lax.bitcast_convert_type(key, jnp.uint32) ^ jnp.uint32(0x80000000)

    def sbody(t, T):
        bit = jnp.uint32(31 - t)
        cand = T | (jnp.uint32(1) << bit)
        cnt = jnp.sum((ukey >= cand).astype(jnp.int32))
        return jnp.where(cnt >= k, cand, T)

    T = jax.lax.fori_loop(0, 32, sbody, jnp.uint32(0))

    kk = jax.lax.bitcast_convert_type(T ^ jnp.uint32(0x80000000), jnp.int32)
    iv = jnp.where(kk < 0, kk ^ 0x7FFFFFFF, kk)
    v = jax.lax.bitcast_convert_type(iv, jnp.float32)
    o_ref[0] = jnp.where(h >= v, jnp.float32(ALPHA), jnp.float32(BETA))


def kernel(inputs):
    b, c, h, w = inputs.shape
    hw = h * w
    lanes = 128
    rows = hw // lanes
    k = int(RATE * hw)
    cc = 16
    n_chunks = c // cc
    x = inputs.reshape(b, c, rows, lanes)
    out = pl.pallas_call(
        functools.partial(_fused_kernel, n_chunks=n_chunks, cc=cc, k=k),
        grid=(b,),
        in_specs=[pl.BlockSpec(memory_space=pltpu.HBM)],
        out_specs=pl.BlockSpec((1, rows, lanes), lambda i: (i, 0, 0)),
        out_shape=jax.ShapeDtypeStruct((b, rows, lanes), jnp.float32),
        scratch_shapes=[
            pltpu.VMEM((rows, lanes), jnp.float32),
            pltpu.VMEM((NBUF, cc, rows, lanes), jnp.float32),
            pltpu.SemaphoreType.DMA((NBUF,)),
        ],
    )(x)
    return out.reshape(b, 1, h, w)
